# baseline (speedup 1.0000x reference)
import functools

import jax
import jax.numpy as jnp
from jax import lax
from jax.experimental import pallas as pl
from jax.experimental.pallas import tpu as pltpu
from jax.experimental.pallas import tpu_sc as plsc

N_NODES = 10000
HID = 512
CHUNK = 128
TN = 400
TB = 512


EB = 80
SB = 16
NROW = 4


def _segsum_sc(x_flat, src2d, dst2d, zeros, n, c_chunks, e_pad):
    mesh = plsc.VectorSubcoreMesh(core_axis_name="c", subcore_axis_name="s")
    n_acc = n + 16
    rows_per_tec = (n_acc // 16) // 8 * 8
    ztail = n_acc - 16 * rows_per_tec
    otail = n - 16 * rows_per_tec
    out_c = 2 if c_chunks == 1 else c_chunks
    if c_chunks == 1:
        edges_per = e_pad // 32
        t_passes = 1
    else:
        edges_per = e_pad // 16
        t_passes = c_chunks // 2
    nb = edges_per // EB
    nsb = nb // SB
    assert nb % SB == 0 and nsb % 2 == 0

    @functools.partial(
        pl.kernel,
        out_type=jax.ShapeDtypeStruct((out_c * n, CHUNK), jnp.float32),
        mesh=mesh,
        scratch_types=[
            pltpu.VMEM_SHARED((n_acc, CHUNK), jnp.float32),
            [pltpu.VMEM((SB, EB), jnp.int32)] * 2,
            [pltpu.VMEM((SB, EB), jnp.int32)] * 2,
            [pltpu.VMEM((EB, CHUNK), jnp.float32)] * NROW,
            [pltpu.SemaphoreType.DMA] * 2,
            [pltpu.SemaphoreType.DMA] * NROW,
            [pltpu.SemaphoreType.DMA] * NROW,
        ],
    )
    def seg_kernel(x_hbm, src_hbm, dst_hbm, z_hbm, out_hbm,
                   acc, isb, dsb, rows, bsem, gsem, ssem):
        cid = lax.axis_index("c")
        sid = lax.axis_index("s")
        r0 = sid * rows_per_tec
        for t in range(t_passes):
            if c_chunks == 1:
                row_start = (cid * (e_pad // 2 // EB)
                             + sid * (edges_per // EB))
                base = None
                out_base = cid * n
            else:
                chunk = cid + 2 * t
                row_start = sid * (edges_per // EB)
                base = chunk * n
                out_base = chunk * n

            def issue_sb(sb, h):
                ro = row_start + sb * SB
                pltpu.async_copy(src_hbm.at[pl.ds(ro, SB)], isb[h], bsem[h])
                pltpu.async_copy(dst_hbm.at[pl.ds(ro, SB)], dsb[h], bsem[h])

            def wait_sb(h):
                pltpu.make_async_copy(src_hbm.at[pl.ds(0, SB)], isb[h],
                                      bsem[h]).wait()
                pltpu.make_async_copy(dst_hbm.at[pl.ds(0, SB)], dsb[h],
                                      bsem[h]).wait()

            def start_gather(h, j, p):
                if base is not None:
                    for v in range(EB // 16):
                        sl = (j, pl.ds(16 * v, 16))
                        isb[h][sl] = isb[h][sl] + base
                pltpu.async_copy(x_hbm.at[isb[h].at[j]], rows[p], gsem[p])

            def wait_gather(p):
                pltpu.make_async_copy(x_hbm.at[pl.ds(0, EB)], rows[p],
                                      gsem[p]).wait()

            def start_scatter(h, j, p):
                pltpu.async_copy(rows[p], acc.at[dsb[h].at[j]], ssem[p],
                                 add=True)

            def wait_scatter(h, j, p):
                pltpu.make_async_copy(rows[p], acc.at[dsb[h].at[j]],
                                      ssem[p]).wait()

            pltpu.sync_copy(z_hbm.at[pl.ds(r0, rows_per_tec)],
                            acc.at[pl.ds(r0, rows_per_tec)])
            if ztail:
                @pl.when(sid == 0)
                def _():
                    pltpu.sync_copy(z_hbm.at[pl.ds(16 * rows_per_tec, ztail)],
                                    acc.at[pl.ds(16 * rows_per_tec, ztail)])
            plsc.subcore_barrier()

            issue_sb(0, 0)

            @pl.loop(0, nsb, step=2)
            def _(sb0):
                for d in range(2):
                    sb = sb0 + d
                    for j in range(SB):
                        i = sb * SB + j
                        p = j % 4
                        if j == 0:
                            wait_sb(d)
                        start_gather(d, j, p)
                        hq, jq = (d, j - 1) if j >= 1 else (1 - d, SB - 1)
                        pq = (j - 1) % 4

                        @pl.when(i >= 1)
                        def _():
                            wait_gather(pq)
                            start_scatter(hq, jq, pq)

                        hr, jr = (d, j - 3) if j >= 3 else (1 - d, SB + j - 3)
                        pr = (j - 3) % 4

                        @pl.when(i >= 3)
                        def _():
                            wait_scatter(hr, jr, pr)

                        if j == 3:
                            @pl.when(sb + 1 < nsb)
                            def _():
                                issue_sb(sb + 1, 1 - d)

            wait_gather((SB - 1) % 4)
            start_scatter(1, SB - 1, (SB - 1) % 4)
            for kk in range(3):
                j = SB - 3 + kk
                wait_scatter(1, j, j % 4)

            plsc.subcore_barrier()
            pltpu.sync_copy(acc.at[pl.ds(r0, rows_per_tec)],
                            out_hbm.at[pl.ds(out_base + r0, rows_per_tec)])
            if otail > 0:
                @pl.when(sid == 0)
                def _():
                    pltpu.sync_copy(
                        acc.at[pl.ds(16 * rows_per_tec, otail)],
                        out_hbm.at[pl.ds(out_base + 16 * rows_per_tec, otail)])
            plsc.subcore_barrier()

    return seg_kernel(x_flat, src2d, dst2d, zeros)


def _pair_gather_sc(gene, dis, i0, i1, b):
    mesh = plsc.VectorSubcoreMesh(core_axis_name="c", subcore_axis_name="s")
    rows_per_w = b // 32
    gb = 128

    @functools.partial(
        pl.kernel,
        out_type=(jax.ShapeDtypeStruct((b, HID), jnp.float32),
                  jax.ShapeDtypeStruct((b, HID), jnp.float32)),
        mesh=mesh,
        scratch_types=[
            pltpu.VMEM((gb,), jnp.int32),
            pltpu.VMEM((gb, HID), jnp.float32),
            pltpu.SemaphoreType.DMA,
        ],
    )
    def gather_kernel(g_hbm, d_hbm, i0_hbm, i1_hbm, o0_hbm, o1_hbm,
                      idx, rows, sem):
        cid = lax.axis_index("c")
        sid = lax.axis_index("s")
        wid = sid * 2 + cid
        base = wid * rows_per_w

        @pl.loop(0, rows_per_w, step=gb)
        def _(g):
            off = base + g
            pltpu.sync_copy(i0_hbm.at[pl.ds(off, gb)], idx)
            pltpu.async_copy(g_hbm.at[idx], rows, sem).wait()
            pltpu.sync_copy(rows, o0_hbm.at[pl.ds(off, gb)])
            pltpu.sync_copy(i1_hbm.at[pl.ds(off, gb)], idx)
            pltpu.async_copy(d_hbm.at[idx], rows, sem).wait()
            pltpu.sync_copy(rows, o1_hbm.at[pl.ds(off, gb)])

    return gather_kernel(gene, dis, i0, i1)


def _layer_tc(agg, h, wrt, wot, b):
    p_parts = agg.shape[0]
    din = h.shape[1]
    n = h.shape[0]
    grid = (n // TN,)

    def body(agg_ref, h_ref, wrt_ref, wot_ref, b_ref, pre_ref, s1_ref, s2_ref):
        i = pl.program_id(0)
        acc = jnp.dot(h_ref[...], wot_ref[...],
                      preferred_element_type=jnp.float32)
        for p in range(p_parts):
            acc += jnp.dot(agg_ref[p], wrt_ref[p],
                           preferred_element_type=jnp.float32)
        pre = jax.nn.leaky_relu(acc + b_ref[...], 0.01)
        pre_ref[...] = pre

        @pl.when(i == 0)
        def _():
            s1_ref[...] = jnp.zeros_like(s1_ref)
            s2_ref[...] = jnp.zeros_like(s2_ref)

        s1_ref[...] += jnp.sum(pre, axis=0, keepdims=True)
        s2_ref[...] += jnp.sum(pre * pre, axis=0, keepdims=True)

    return pl.pallas_call(
        body,
        grid=grid,
        in_specs=[
            pl.BlockSpec((p_parts, TN, CHUNK), lambda i: (0, i, 0)),
            pl.BlockSpec((TN, din), lambda i: (i, 0)),
            pl.BlockSpec((p_parts, CHUNK, HID), lambda i: (0, 0, 0)),
            pl.BlockSpec((din, HID), lambda i: (0, 0)),
            pl.BlockSpec((1, HID), lambda i: (0, 0)),
        ],
        out_specs=[
            pl.BlockSpec((TN, HID), lambda i: (i, 0)),
            pl.BlockSpec((1, HID), lambda i: (0, 0)),
            pl.BlockSpec((1, HID), lambda i: (0, 0)),
        ],
        out_shape=[
            jax.ShapeDtypeStruct((n, HID), jnp.float32),
            jax.ShapeDtypeStruct((1, HID), jnp.float32),
            jax.ShapeDtypeStruct((1, HID), jnp.float32),
        ],
    )(agg, h, wrt, wot, b)


def _norm_tc(pre, s1, s2, gamma, beta, n, emit_chunks):
    nc = HID // CHUNK

    def body(pre_ref, s1_ref, s2_ref, g_ref, b_ref, hn_ref, *maybe_hc):
        mu = s1_ref[...] * (1.0 / n)
        var = s2_ref[...] * (1.0 / n) - mu * mu
        a = g_ref[...] * lax.rsqrt(var + 1e-5)
        c = b_ref[...] - mu * a
        hn = pre_ref[...] * a + c
        hn_ref[...] = hn
        if emit_chunks:
            hc_ref = maybe_hc[0]
            for j in range(nc):
                hc_ref[j] = hn[:, j * CHUNK:(j + 1) * CHUNK]

    out_specs = [pl.BlockSpec((TN, HID), lambda i: (i, 0))]
    out_shape = [jax.ShapeDtypeStruct((n, HID), jnp.float32)]
    if emit_chunks:
        out_specs.append(pl.BlockSpec((nc, TN, CHUNK), lambda i: (0, i, 0)))
        out_shape.append(jax.ShapeDtypeStruct((nc, n, CHUNK), jnp.float32))

    return pl.pallas_call(
        body,
        grid=(n // TN,),
        in_specs=[
            pl.BlockSpec((TN, HID), lambda i: (i, 0)),
            pl.BlockSpec((1, HID), lambda i: (0, 0)),
            pl.BlockSpec((1, HID), lambda i: (0, 0)),
            pl.BlockSpec((1, HID), lambda i: (0, 0)),
            pl.BlockSpec((1, HID), lambda i: (0, 0)),
        ],
        out_specs=out_specs,
        out_shape=out_shape,
    )(pre, s1, s2, gamma, beta)


def _norm_mix_tc(pre2, s1, s2, gamma, beta, h0, h1, n):
    def body(pre_ref, s1_ref, s2_ref, g_ref, b_ref, h0_ref, h1_ref, out_ref):
        mu = s1_ref[...] * (1.0 / n)
        var = s2_ref[...] * (1.0 / n) - mu * mu
        a = g_ref[...] * lax.rsqrt(var + 1e-5)
        c = b_ref[...] - mu * a
        h2 = pre_ref[...] * a + c
        out_ref[...] = 0.7 * h0_ref[...] + 0.2 * h1_ref[...] + 0.1 * h2

    return pl.pallas_call(
        body,
        grid=(n // TN,),
        in_specs=[
            pl.BlockSpec((TN, HID), lambda i: (i, 0)),
            pl.BlockSpec((1, HID), lambda i: (0, 0)),
            pl.BlockSpec((1, HID), lambda i: (0, 0)),
            pl.BlockSpec((1, HID), lambda i: (0, 0)),
            pl.BlockSpec((1, HID), lambda i: (0, 0)),
            pl.BlockSpec((TN, HID), lambda i: (i, 0)),
            pl.BlockSpec((TN, HID), lambda i: (i, 0)),
        ],
        out_specs=pl.BlockSpec((TN, HID), lambda i: (i, 0)),
        out_shape=jax.ShapeDtypeStruct((n, HID), jnp.float32),
    )(pre2, s1, s2, gamma, beta, h0, h1)


def _mlp_tc(x0, x1, w1a, w1b, b1, w2, b2, w3, b3, w4, b4):
    b = x0.shape[0]

    def body(x0_ref, x1_ref, w1a_ref, w1b_ref, b1_ref, w2_ref, b2_ref,
             w3_ref, b3_ref, w4_ref, b4_ref, out_ref):
        bf = jnp.bfloat16
        h1 = jnp.dot(x0_ref[...].astype(bf), w1a_ref[...].astype(bf),
                     preferred_element_type=jnp.float32)
        h1 += jnp.dot(x1_ref[...].astype(bf), w1b_ref[...].astype(bf),
                      preferred_element_type=jnp.float32)
        h1 = jax.nn.leaky_relu(h1 + b1_ref[...], 0.01)
        h2 = jax.nn.leaky_relu(
            jnp.dot(h1.astype(bf), w2_ref[...].astype(bf),
                    preferred_element_type=jnp.float32)
            + b2_ref[...], 0.01)
        h3 = jax.nn.leaky_relu(
            jnp.dot(h2, w3_ref[...], preferred_element_type=jnp.float32)
            + b3_ref[...], 0.01)
        out_ref[...] = (jnp.dot(h3, w4_ref[...],
                                preferred_element_type=jnp.float32)
                        + b4_ref[...])

    d1, d2, d3, d4 = 2048, 1024, 512, 2
    return pl.pallas_call(
        body,
        grid=(b // TB,),
        in_specs=[
            pl.BlockSpec((TB, HID), lambda i: (i, 0)),
            pl.BlockSpec((TB, HID), lambda i: (i, 0)),
            pl.BlockSpec((HID, d1), lambda i: (0, 0)),
            pl.BlockSpec((HID, d1), lambda i: (0, 0)),
            pl.BlockSpec((1, d1), lambda i: (0, 0)),
            pl.BlockSpec((d1, d2), lambda i: (0, 0)),
            pl.BlockSpec((1, d2), lambda i: (0, 0)),
            pl.BlockSpec((d2, d3), lambda i: (0, 0)),
            pl.BlockSpec((1, d3), lambda i: (0, 0)),
            pl.BlockSpec((d3, d4), lambda i: (0, 0)),
            pl.BlockSpec((1, d4), lambda i: (0, 0)),
        ],
        out_specs=pl.BlockSpec((TB, d4), lambda i: (i, 0)),
        out_shape=jax.ShapeDtypeStruct((b, d4), jnp.float32),
    )(x0, x1, w1a, w1b, b1, w2, b2, w3, b3, w4, b4)


def _pad_edges(ei, n):
    e = ei.shape[1]
    unit = 32 * EB * SB
    e_pad = -(-e // unit) * unit
    pad = e_pad - e
    r = jnp.arange(pad, dtype=jnp.int32)
    src = jnp.concatenate([ei[0], r % n]).reshape(e_pad // EB, EB)
    dst = jnp.concatenate([ei[1], n + (r % 16)]).reshape(e_pad // EB, EB)
    return src, dst, e_pad


def _graph_chain(x, src, dst, e_pad, params, prefix, zeros):
    p = params
    n = x.shape[0]
    hs = []
    c_chunks = 1
    h_flat = x
    h_mat = x
    for k in range(3):
        name = f"{prefix}{k}"
        w_rel = p["W_rel_" + name]
        w_root = p["W_root_" + name]
        bias = p["b_" + name].reshape(1, HID)
        gamma = p["gamma_" + name].reshape(1, HID)
        beta = p["beta_" + name].reshape(1, HID)

        agg = _segsum_sc(h_flat, src, dst, zeros, n, c_chunks, e_pad)
        p_parts = 2 if c_chunks == 1 else c_chunks
        if c_chunks == 1:
            wrt_s = jnp.stack([w_rel.T, w_rel.T])
        else:
            wrt_s = w_rel.T.reshape(p_parts, CHUNK, HID)
        agg_r = agg.reshape(p_parts, n, CHUNK)
        pre, s1, s2 = _layer_tc(agg_r, h_mat, wrt_s, w_root.T, bias)
        if k < 2:
            h_mat, hc = _norm_tc(pre, s1, s2, gamma, beta, n, True)
            c_chunks = HID // CHUNK
            h_flat = hc.reshape(c_chunks * n, CHUNK)
            hs.append(h_mat)
        else:
            out = _norm_mix_tc(pre, s1, s2, gamma, beta, hs[0], hs[1], n)
    return out


def kernel(gene_x, disease_x, params, gene_edge_index, disease_edge_index,
           batch_idx):
    p = params
    n = gene_x.shape[0]
    zeros = jnp.zeros((n + 16, CHUNK), jnp.float32)
    gsrc, gdst, e_g = _pad_edges(gene_edge_index, n)
    dsrc, ddst, e_d = _pad_edges(disease_edge_index, n)

    gene_out = _graph_chain(gene_x, gsrc, gdst, e_g, p, "g", zeros)
    dis_out = _graph_chain(disease_x, dsrc, ddst, e_d, p, "d", zeros)

    i0 = batch_idx[:, 0]
    i1 = batch_idx[:, 1]
    x0, x1 = _pair_gather_sc(gene_out, dis_out, i0, i1, batch_idx.shape[0])

    w1t = p["W_lin1"].T
    w1a = w1t[:HID]
    w1b = w1t[HID:]
    return _mlp_tc(
        x0, x1, w1a, w1b, p["b_lin1"].reshape(1, -1),
        p["W_lin2"].T, p["b_lin2"].reshape(1, -1),
        p["W_lin3"].T, p["b_lin3"].reshape(1, -1),
        p["W_lin4"].T, p["b_lin4"].reshape(1, -1),
    )


# speedup vs baseline: 1.0016x; 1.0016x over previous
import functools

import jax
import jax.numpy as jnp
from jax import lax
from jax.experimental import pallas as pl
from jax.experimental.pallas import tpu as pltpu
from jax.experimental.pallas import tpu_sc as plsc

N_NODES = 10000
HID = 512
CHUNK = 128
TN = 400
TB = 512


EB = 80
SB = 16
NROW = 4


def _segsum_sc(x_flat, src2d, dst2d, zeros, n, c_chunks, e_pad):
    mesh = plsc.VectorSubcoreMesh(core_axis_name="c", subcore_axis_name="s")
    n_acc = n + 16
    rows_per_tec = (n_acc // 16) // 8 * 8
    ztail = n_acc - 16 * rows_per_tec
    otail = n - 16 * rows_per_tec
    out_c = 2 if c_chunks == 1 else c_chunks
    if c_chunks == 1:
        edges_per = e_pad // 32
        t_passes = 1
    else:
        edges_per = e_pad // 16
        t_passes = c_chunks // 2
    nb = edges_per // EB
    nsb = nb // SB
    assert nb % SB == 0 and nsb % 2 == 0

    @functools.partial(
        pl.kernel,
        out_type=jax.ShapeDtypeStruct((out_c * n, CHUNK), jnp.float32),
        mesh=mesh,
        scratch_types=[
            pltpu.VMEM_SHARED((n_acc, CHUNK), jnp.float32),
            [pltpu.VMEM((SB, EB), jnp.int32)] * 2,
            [pltpu.VMEM((SB, EB), jnp.int32)] * 2,
            [pltpu.VMEM((EB, CHUNK), jnp.float32)] * NROW,
            [pltpu.SemaphoreType.DMA] * 2,
            [pltpu.SemaphoreType.DMA] * NROW,
            [pltpu.SemaphoreType.DMA] * NROW,
        ],
    )
    def seg_kernel(x_hbm, src_hbm, dst_hbm, z_hbm, out_hbm,
                   acc, isb, dsb, rows, bsem, gsem, ssem):
        cid = lax.axis_index("c")
        sid = lax.axis_index("s")
        r0 = sid * rows_per_tec
        for t in range(t_passes):
            if c_chunks == 1:
                row_start = (cid * (e_pad // 2 // EB)
                             + sid * (edges_per // EB))
                base = None
                out_base = cid * n
            else:
                chunk = cid + 2 * t
                row_start = sid * (edges_per // EB)
                base = chunk * n
                out_base = chunk * n

            def issue_sb(sb, h):
                ro = row_start + sb * SB
                pltpu.async_copy(src_hbm.at[pl.ds(ro, SB)], isb[h], bsem[h])
                pltpu.async_copy(dst_hbm.at[pl.ds(ro, SB)], dsb[h], bsem[h])

            def wait_sb(h):
                pltpu.make_async_copy(src_hbm.at[pl.ds(0, SB)], isb[h],
                                      bsem[h]).wait()
                pltpu.make_async_copy(dst_hbm.at[pl.ds(0, SB)], dsb[h],
                                      bsem[h]).wait()

            def start_gather(h, j, p):
                if base is not None:
                    for v in range(EB // 16):
                        sl = (j, pl.ds(16 * v, 16))
                        isb[h][sl] = isb[h][sl] + base
                pltpu.async_copy(x_hbm.at[isb[h].at[j]], rows[p], gsem[p])

            def wait_gather(p):
                pltpu.make_async_copy(x_hbm.at[pl.ds(0, EB)], rows[p],
                                      gsem[p]).wait()

            def start_scatter(h, j, p):
                pltpu.async_copy(rows[p], acc.at[dsb[h].at[j]], ssem[p],
                                 add=True)

            def wait_scatter(h, j, p):
                pltpu.make_async_copy(rows[p], acc.at[dsb[h].at[j]],
                                      ssem[p]).wait()

            pltpu.sync_copy(z_hbm.at[pl.ds(r0, rows_per_tec)],
                            acc.at[pl.ds(r0, rows_per_tec)])
            if ztail:
                @pl.when(sid == 0)
                def _():
                    pltpu.sync_copy(z_hbm.at[pl.ds(16 * rows_per_tec, ztail)],
                                    acc.at[pl.ds(16 * rows_per_tec, ztail)])
            plsc.subcore_barrier()

            issue_sb(0, 0)

            @pl.loop(0, nsb, step=2)
            def _(sb0):
                for d in range(2):
                    sb = sb0 + d
                    for j in range(SB):
                        i = sb * SB + j
                        p = j % 4
                        if j == 0:
                            wait_sb(d)
                        start_gather(d, j, p)
                        hq, jq = (d, j - 1) if j >= 1 else (1 - d, SB - 1)
                        pq = (j - 1) % 4

                        @pl.when(i >= 1)
                        def _():
                            wait_gather(pq)
                            start_scatter(hq, jq, pq)

                        hr, jr = (d, j - 3) if j >= 3 else (1 - d, SB + j - 3)
                        pr = (j - 3) % 4

                        @pl.when(i >= 3)
                        def _():
                            wait_scatter(hr, jr, pr)

                        if j == 3:
                            @pl.when(sb + 1 < nsb)
                            def _():
                                issue_sb(sb + 1, 1 - d)

            wait_gather((SB - 1) % 4)
            start_scatter(1, SB - 1, (SB - 1) % 4)
            for kk in range(3):
                j = SB - 3 + kk
                wait_scatter(1, j, j % 4)

            plsc.subcore_barrier()
            pltpu.sync_copy(acc.at[pl.ds(r0, rows_per_tec)],
                            out_hbm.at[pl.ds(out_base + r0, rows_per_tec)])
            if otail > 0:
                @pl.when(sid == 0)
                def _():
                    pltpu.sync_copy(
                        acc.at[pl.ds(16 * rows_per_tec, otail)],
                        out_hbm.at[pl.ds(out_base + 16 * rows_per_tec, otail)])
            plsc.subcore_barrier()

    return seg_kernel(x_flat, src2d, dst2d, zeros)


def _pair_gather_sc(gene, dis, i0, i1, b):
    mesh = plsc.VectorSubcoreMesh(core_axis_name="c", subcore_axis_name="s")
    rows_per_w = b // 32
    gb = 64

    @functools.partial(
        pl.kernel,
        out_type=(jax.ShapeDtypeStruct((b, HID), jnp.float32),
                  jax.ShapeDtypeStruct((b, HID), jnp.float32)),
        mesh=mesh,
        scratch_types=[
            [pltpu.VMEM((gb,), jnp.int32)] * 2,
            [pltpu.VMEM((gb, HID), jnp.float32)] * 2,
            [pltpu.SemaphoreType.DMA] * 2,
            [pltpu.SemaphoreType.DMA] * 2,
            [pltpu.SemaphoreType.DMA] * 2,
        ],
    )
    def gather_kernel(g_hbm, d_hbm, i0_hbm, i1_hbm, o0_hbm, o1_hbm,
                      idx, rows, isem, gsem, wsem):
        cid = lax.axis_index("c")
        sid = lax.axis_index("s")
        wid = sid * 2 + cid
        base = wid * rows_per_w
        items = []
        for g in range(rows_per_w // gb):
            items.append((g_hbm, i0_hbm, o0_hbm, base + g * gb))
            items.append((d_hbm, i1_hbm, o1_hbm, base + g * gb))
        ni = len(items)

        def issue_idx(i):
            _, isrc, _, off = items[i]
            pltpu.async_copy(isrc.at[pl.ds(off, gb)], idx[i % 2],
                             isem[i % 2])

        issue_idx(0)
        issue_idx(1)
        for i in range(ni):
            p = i % 2
            tab, isrc, out, off = items[i]
            pltpu.make_async_copy(isrc.at[pl.ds(off, gb)], idx[p],
                                  isem[p]).wait()
            if i >= 2:
                prev = items[i - 2]
                pltpu.make_async_copy(rows[p], prev[2].at[pl.ds(prev[3], gb)],
                                      wsem[p]).wait()
            pltpu.async_copy(tab.at[idx[p]], rows[p], gsem[p])
            pltpu.make_async_copy(tab.at[pl.ds(0, gb)], rows[p],
                                  gsem[p]).wait()
            pltpu.async_copy(rows[p], out.at[pl.ds(off, gb)], wsem[p])
            if i + 2 < ni:
                issue_idx(i + 2)
        for i in range(ni - 2, ni):
            p = i % 2
            pltpu.make_async_copy(rows[p], items[i][2].at[pl.ds(items[i][3],
                                                                gb)],
                                  wsem[p]).wait()

    return gather_kernel(gene, dis, i0, i1)


def _layer_tc(agg, h, wrt, wot, b):
    p_parts = agg.shape[0]
    din = h.shape[1]
    n = h.shape[0]
    grid = (n // TN,)

    def body(agg_ref, h_ref, wrt_ref, wot_ref, b_ref, pre_ref, s1_ref, s2_ref):
        i = pl.program_id(0)
        acc = jnp.dot(h_ref[...], wot_ref[...],
                      preferred_element_type=jnp.float32)
        for p in range(p_parts):
            acc += jnp.dot(agg_ref[p], wrt_ref[p],
                           preferred_element_type=jnp.float32)
        pre = jax.nn.leaky_relu(acc + b_ref[...], 0.01)
        pre_ref[...] = pre

        @pl.when(i == 0)
        def _():
            s1_ref[...] = jnp.zeros_like(s1_ref)
            s2_ref[...] = jnp.zeros_like(s2_ref)

        s1_ref[...] += jnp.sum(pre, axis=0, keepdims=True)
        s2_ref[...] += jnp.sum(pre * pre, axis=0, keepdims=True)

    return pl.pallas_call(
        body,
        grid=grid,
        in_specs=[
            pl.BlockSpec((p_parts, TN, CHUNK), lambda i: (0, i, 0)),
            pl.BlockSpec((TN, din), lambda i: (i, 0)),
            pl.BlockSpec((p_parts, CHUNK, HID), lambda i: (0, 0, 0)),
            pl.BlockSpec((din, HID), lambda i: (0, 0)),
            pl.BlockSpec((1, HID), lambda i: (0, 0)),
        ],
        out_specs=[
            pl.BlockSpec((TN, HID), lambda i: (i, 0)),
            pl.BlockSpec((1, HID), lambda i: (0, 0)),
            pl.BlockSpec((1, HID), lambda i: (0, 0)),
        ],
        out_shape=[
            jax.ShapeDtypeStruct((n, HID), jnp.float32),
            jax.ShapeDtypeStruct((1, HID), jnp.float32),
            jax.ShapeDtypeStruct((1, HID), jnp.float32),
        ],
    )(agg, h, wrt, wot, b)


def _norm_tc(pre, s1, s2, gamma, beta, n, emit_chunks):
    nc = HID // CHUNK

    def body(pre_ref, s1_ref, s2_ref, g_ref, b_ref, hn_ref, *maybe_hc):
        mu = s1_ref[...] * (1.0 / n)
        var = s2_ref[...] * (1.0 / n) - mu * mu
        a = g_ref[...] * lax.rsqrt(var + 1e-5)
        c = b_ref[...] - mu * a
        hn = pre_ref[...] * a + c
        hn_ref[...] = hn
        if emit_chunks:
            hc_ref = maybe_hc[0]
            for j in range(nc):
                hc_ref[j] = hn[:, j * CHUNK:(j + 1) * CHUNK]

    out_specs = [pl.BlockSpec((TN, HID), lambda i: (i, 0))]
    out_shape = [jax.ShapeDtypeStruct((n, HID), jnp.float32)]
    if emit_chunks:
        out_specs.append(pl.BlockSpec((nc, TN, CHUNK), lambda i: (0, i, 0)))
        out_shape.append(jax.ShapeDtypeStruct((nc, n, CHUNK), jnp.float32))

    return pl.pallas_call(
        body,
        grid=(n // TN,),
        in_specs=[
            pl.BlockSpec((TN, HID), lambda i: (i, 0)),
            pl.BlockSpec((1, HID), lambda i: (0, 0)),
            pl.BlockSpec((1, HID), lambda i: (0, 0)),
            pl.BlockSpec((1, HID), lambda i: (0, 0)),
            pl.BlockSpec((1, HID), lambda i: (0, 0)),
        ],
        out_specs=out_specs,
        out_shape=out_shape,
    )(pre, s1, s2, gamma, beta)


def _norm_mix_tc(pre2, s1, s2, gamma, beta, h0, h1, n):
    def body(pre_ref, s1_ref, s2_ref, g_ref, b_ref, h0_ref, h1_ref, out_ref):
        mu = s1_ref[...] * (1.0 / n)
        var = s2_ref[...] * (1.0 / n) - mu * mu
        a = g_ref[...] * lax.rsqrt(var + 1e-5)
        c = b_ref[...] - mu * a
        h2 = pre_ref[...] * a + c
        out_ref[...] = 0.7 * h0_ref[...] + 0.2 * h1_ref[...] + 0.1 * h2

    return pl.pallas_call(
        body,
        grid=(n // TN,),
        in_specs=[
            pl.BlockSpec((TN, HID), lambda i: (i, 0)),
            pl.BlockSpec((1, HID), lambda i: (0, 0)),
            pl.BlockSpec((1, HID), lambda i: (0, 0)),
            pl.BlockSpec((1, HID), lambda i: (0, 0)),
            pl.BlockSpec((1, HID), lambda i: (0, 0)),
            pl.BlockSpec((TN, HID), lambda i: (i, 0)),
            pl.BlockSpec((TN, HID), lambda i: (i, 0)),
        ],
        out_specs=pl.BlockSpec((TN, HID), lambda i: (i, 0)),
        out_shape=jax.ShapeDtypeStruct((n, HID), jnp.float32),
    )(pre2, s1, s2, gamma, beta, h0, h1)


def _mlp_tc(x0, x1, w1a, w1b, b1, w2, b2, w3, b3, w4, b4):
    b = x0.shape[0]

    def body(x0_ref, x1_ref, w1a_ref, w1b_ref, b1_ref, w2_ref, b2_ref,
             w3_ref, b3_ref, w4_ref, b4_ref, out_ref):
        bf = jnp.bfloat16
        h1 = jnp.dot(x0_ref[...].astype(bf), w1a_ref[...].astype(bf),
                     preferred_element_type=jnp.float32)
        h1 += jnp.dot(x1_ref[...].astype(bf), w1b_ref[...].astype(bf),
                      preferred_element_type=jnp.float32)
        h1 = jax.nn.leaky_relu(h1 + b1_ref[...], 0.01)
        h2 = jax.nn.leaky_relu(
            jnp.dot(h1.astype(bf), w2_ref[...].astype(bf),
                    preferred_element_type=jnp.float32)
            + b2_ref[...], 0.01)
        h3 = jax.nn.leaky_relu(
            jnp.dot(h2.astype(bf), w3_ref[...].astype(bf),
                    preferred_element_type=jnp.float32)
            + b3_ref[...], 0.01)
        out_ref[...] = (jnp.dot(h3, w4_ref[...],
                                preferred_element_type=jnp.float32)
                        + b4_ref[...])

    d1, d2, d3, d4 = 2048, 1024, 512, 2
    return pl.pallas_call(
        body,
        grid=(b // TB,),
        in_specs=[
            pl.BlockSpec((TB, HID), lambda i: (i, 0)),
            pl.BlockSpec((TB, HID), lambda i: (i, 0)),
            pl.BlockSpec((HID, d1), lambda i: (0, 0)),
            pl.BlockSpec((HID, d1), lambda i: (0, 0)),
            pl.BlockSpec((1, d1), lambda i: (0, 0)),
            pl.BlockSpec((d1, d2), lambda i: (0, 0)),
            pl.BlockSpec((1, d2), lambda i: (0, 0)),
            pl.BlockSpec((d2, d3), lambda i: (0, 0)),
            pl.BlockSpec((1, d3), lambda i: (0, 0)),
            pl.BlockSpec((d3, d4), lambda i: (0, 0)),
            pl.BlockSpec((1, d4), lambda i: (0, 0)),
        ],
        out_specs=pl.BlockSpec((TB, d4), lambda i: (i, 0)),
        out_shape=jax.ShapeDtypeStruct((b, d4), jnp.float32),
    )(x0, x1, w1a, w1b, b1, w2, b2, w3, b3, w4, b4)


def _pad_edges(ei, n):
    e = ei.shape[1]
    unit = 32 * EB * SB
    e_pad = -(-e // unit) * unit
    pad = e_pad - e
    r = jnp.arange(pad, dtype=jnp.int32)
    src = jnp.concatenate([ei[0], r % n]).reshape(e_pad // EB, EB)
    dst = jnp.concatenate([ei[1], n + (r % 16)]).reshape(e_pad // EB, EB)
    return src, dst, e_pad


def _graph_chain(x, src, dst, e_pad, params, prefix, zeros):
    p = params
    n = x.shape[0]
    hs = []
    c_chunks = 1
    h_flat = x
    h_mat = x
    for k in range(3):
        name = f"{prefix}{k}"
        w_rel = p["W_rel_" + name]
        w_root = p["W_root_" + name]
        bias = p["b_" + name].reshape(1, HID)
        gamma = p["gamma_" + name].reshape(1, HID)
        beta = p["beta_" + name].reshape(1, HID)

        agg = _segsum_sc(h_flat, src, dst, zeros, n, c_chunks, e_pad)
        p_parts = 2 if c_chunks == 1 else c_chunks
        if c_chunks == 1:
            wrt_s = jnp.stack([w_rel.T, w_rel.T])
        else:
            wrt_s = w_rel.T.reshape(p_parts, CHUNK, HID)
        agg_r = agg.reshape(p_parts, n, CHUNK)
        pre, s1, s2 = _layer_tc(agg_r, h_mat, wrt_s, w_root.T, bias)
        if k < 2:
            h_mat, hc = _norm_tc(pre, s1, s2, gamma, beta, n, True)
            c_chunks = HID // CHUNK
            h_flat = hc.reshape(c_chunks * n, CHUNK)
            hs.append(h_mat)
        else:
            out = _norm_mix_tc(pre, s1, s2, gamma, beta, hs[0], hs[1], n)
    return out


def kernel(gene_x, disease_x, params, gene_edge_index, disease_edge_index,
           batch_idx):
    p = params
    n = gene_x.shape[0]
    zeros = jnp.zeros((n + 16, CHUNK), jnp.float32)
    gsrc, gdst, e_g = _pad_edges(gene_edge_index, n)
    dsrc, ddst, e_d = _pad_edges(disease_edge_index, n)

    gene_out = _graph_chain(gene_x, gsrc, gdst, e_g, p, "g", zeros)
    dis_out = _graph_chain(disease_x, dsrc, ddst, e_d, p, "d", zeros)

    bt = batch_idx.T
    i0 = bt[0]
    i1 = bt[1]
    x0, x1 = _pair_gather_sc(gene_out, dis_out, i0, i1, batch_idx.shape[0])

    w1t = p["W_lin1"].T
    w1a = w1t[:HID]
    w1b = w1t[HID:]
    return _mlp_tc(
        x0, x1, w1a, w1b, p["b_lin1"].reshape(1, -1),
        p["W_lin2"].T, p["b_lin2"].reshape(1, -1),
        p["W_lin3"].T, p["b_lin3"].reshape(1, -1),
        p["W_lin4"].T, p["b_lin4"].reshape(1, -1),
    )


# speedup vs baseline: 1.1026x; 1.1009x over previous
import functools

import jax
import jax.numpy as jnp
from jax import lax
from jax.experimental import pallas as pl
from jax.experimental.pallas import tpu as pltpu
from jax.experimental.pallas import tpu_sc as plsc

N_NODES = 10000
HID = 512
CHUNK = 128
TN = 400
TB = 512


EB = 80
SB = 16
NROW = 4


def _segsum_sc(x_flat, src2d, dst2d, zeros, n, c_chunks, e_pad):
    mesh = plsc.VectorSubcoreMesh(core_axis_name="c", subcore_axis_name="s")
    n_acc = n + 16
    rows_per_tec = (n_acc // 16) // 8 * 8
    ztail = n_acc - 16 * rows_per_tec
    otail = n - 16 * rows_per_tec
    out_c = 2 if c_chunks == 1 else c_chunks
    if c_chunks == 1:
        edges_per = e_pad // 32
        t_passes = 1
    else:
        edges_per = e_pad // 16
        t_passes = c_chunks // 2
    nb = edges_per // EB
    nsb = nb // SB
    assert nb % SB == 0 and nsb % 2 == 0

    @functools.partial(
        pl.kernel,
        out_type=jax.ShapeDtypeStruct((out_c * n, CHUNK), jnp.float32),
        mesh=mesh,
        scratch_types=[
            pltpu.VMEM_SHARED((n_acc, CHUNK), jnp.float32),
            [pltpu.VMEM((SB, EB), jnp.int32)] * 2,
            [pltpu.VMEM((SB, EB), jnp.int32)] * 2,
            [pltpu.VMEM((EB, CHUNK), jnp.float32)] * NROW,
            [pltpu.SemaphoreType.DMA] * 2,
            [pltpu.SemaphoreType.DMA] * NROW,
            [pltpu.SemaphoreType.DMA] * NROW,
        ],
    )
    def seg_kernel(x_hbm, src_hbm, dst_hbm, z_hbm, out_hbm,
                   acc, isb, dsb, rows, bsem, gsem, ssem):
        cid = lax.axis_index("c")
        sid = lax.axis_index("s")
        r0 = sid * rows_per_tec
        for t in range(t_passes):
            if c_chunks == 1:
                row_start = (cid * (e_pad // 2 // EB)
                             + sid * (edges_per // EB))
                base = None
                out_base = cid * n
            else:
                chunk = cid + 2 * t
                row_start = sid * (edges_per // EB)
                base = chunk * n
                out_base = chunk * n

            def issue_sb(sb, h):
                ro = row_start + sb * SB
                pltpu.async_copy(src_hbm.at[pl.ds(ro, SB)], isb[h], bsem[h])
                pltpu.async_copy(dst_hbm.at[pl.ds(ro, SB)], dsb[h], bsem[h])

            def wait_sb(h):
                pltpu.make_async_copy(src_hbm.at[pl.ds(0, SB)], isb[h],
                                      bsem[h]).wait()
                pltpu.make_async_copy(dst_hbm.at[pl.ds(0, SB)], dsb[h],
                                      bsem[h]).wait()

            def start_gather(h, j, p):
                if base is not None:
                    for v in range(EB // 16):
                        sl = (j, pl.ds(16 * v, 16))
                        isb[h][sl] = isb[h][sl] + base
                pltpu.async_copy(x_hbm.at[isb[h].at[j]], rows[p], gsem[p])

            def wait_gather(p):
                pltpu.make_async_copy(x_hbm.at[pl.ds(0, EB)], rows[p],
                                      gsem[p]).wait()

            def start_scatter(h, j, p):
                pltpu.async_copy(rows[p], acc.at[dsb[h].at[j]], ssem[p],
                                 add=True)

            def wait_scatter(h, j, p):
                pltpu.make_async_copy(rows[p], acc.at[dsb[h].at[j]],
                                      ssem[p]).wait()

            pltpu.sync_copy(z_hbm.at[pl.ds(r0, rows_per_tec)],
                            acc.at[pl.ds(r0, rows_per_tec)])
            if ztail:
                @pl.when(sid == 0)
                def _():
                    pltpu.sync_copy(z_hbm.at[pl.ds(16 * rows_per_tec, ztail)],
                                    acc.at[pl.ds(16 * rows_per_tec, ztail)])
            plsc.subcore_barrier()

            issue_sb(0, 0)

            @pl.loop(0, nsb, step=2)
            def _(sb0):
                for d in range(2):
                    sb = sb0 + d
                    for j in range(SB):
                        i = sb * SB + j
                        p = j % 4
                        if j == 0:
                            wait_sb(d)
                        hr, jr = (d, j - 4) if j >= 4 else (1 - d, SB + j - 4)

                        @pl.when(i >= 4)
                        def _():
                            wait_scatter(hr, jr, p)

                        start_gather(d, j, p)
                        hq, jq = (d, j - 2) if j >= 2 else (1 - d, SB + j - 2)
                        pq = (j - 2) % 4

                        @pl.when(i >= 2)
                        def _():
                            wait_gather(pq)
                            start_scatter(hq, jq, pq)

                        if j == 3:
                            @pl.when(sb + 1 < nsb)
                            def _():
                                issue_sb(sb + 1, 1 - d)

            for j in (SB - 2, SB - 1):
                wait_gather(j % 4)
                start_scatter(1, j, j % 4)
            for j in range(SB - 4, SB):
                wait_scatter(1, j, j % 4)

            plsc.subcore_barrier()
            pltpu.sync_copy(acc.at[pl.ds(r0, rows_per_tec)],
                            out_hbm.at[pl.ds(out_base + r0, rows_per_tec)])
            if otail > 0:
                @pl.when(sid == 0)
                def _():
                    pltpu.sync_copy(
                        acc.at[pl.ds(16 * rows_per_tec, otail)],
                        out_hbm.at[pl.ds(out_base + 16 * rows_per_tec, otail)])
            plsc.subcore_barrier()

    return seg_kernel(x_flat, src2d, dst2d, zeros)


def _pair_gather_sc(gene, dis, i0, i1, b):
    mesh = plsc.VectorSubcoreMesh(core_axis_name="c", subcore_axis_name="s")
    rows_per_w = b // 32
    gb = 64

    @functools.partial(
        pl.kernel,
        out_type=(jax.ShapeDtypeStruct((b, HID), jnp.float32),
                  jax.ShapeDtypeStruct((b, HID), jnp.float32)),
        mesh=mesh,
        scratch_types=[
            [pltpu.VMEM((gb,), jnp.int32)] * 2,
            [pltpu.VMEM((gb, HID), jnp.float32)] * 2,
            [pltpu.SemaphoreType.DMA] * 2,
            [pltpu.SemaphoreType.DMA] * 2,
            [pltpu.SemaphoreType.DMA] * 2,
        ],
    )
    def gather_kernel(g_hbm, d_hbm, i0_hbm, i1_hbm, o0_hbm, o1_hbm,
                      idx, rows, isem, gsem, wsem):
        cid = lax.axis_index("c")
        sid = lax.axis_index("s")
        wid = sid * 2 + cid
        base = wid * rows_per_w
        items = []
        for g in range(rows_per_w // gb):
            items.append((g_hbm, i0_hbm, o0_hbm, base + g * gb))
            items.append((d_hbm, i1_hbm, o1_hbm, base + g * gb))
        ni = len(items)

        def issue_idx(i):
            _, isrc, _, off = items[i]
            pltpu.async_copy(isrc.at[pl.ds(off, gb)], idx[i % 2],
                             isem[i % 2])

        issue_idx(0)
        issue_idx(1)
        for i in range(ni):
            p = i % 2
            tab, isrc, out, off = items[i]
            pltpu.make_async_copy(isrc.at[pl.ds(off, gb)], idx[p],
                                  isem[p]).wait()
            if i >= 2:
                prev = items[i - 2]
                pltpu.make_async_copy(rows[p], prev[2].at[pl.ds(prev[3], gb)],
                                      wsem[p]).wait()
            pltpu.async_copy(tab.at[idx[p]], rows[p], gsem[p])
            pltpu.make_async_copy(tab.at[pl.ds(0, gb)], rows[p],
                                  gsem[p]).wait()
            pltpu.async_copy(rows[p], out.at[pl.ds(off, gb)], wsem[p])
            if i + 2 < ni:
                issue_idx(i + 2)
        for i in range(ni - 2, ni):
            p = i % 2
            pltpu.make_async_copy(rows[p], items[i][2].at[pl.ds(items[i][3],
                                                                gb)],
                                  wsem[p]).wait()

    return gather_kernel(gene, dis, i0, i1)


def _layer_tc(agg, h, wrt, wot, b):
    p_parts = agg.shape[0]
    din = h.shape[1]
    n = h.shape[0]
    grid = (n // TN,)

    def body(agg_ref, h_ref, wrt_ref, wot_ref, b_ref, pre_ref, s1_ref, s2_ref):
        i = pl.program_id(0)
        acc = jnp.dot(h_ref[...], wot_ref[...],
                      preferred_element_type=jnp.float32)
        for p in range(p_parts):
            acc += jnp.dot(agg_ref[p], wrt_ref[p],
                           preferred_element_type=jnp.float32)
        pre = jax.nn.leaky_relu(acc + b_ref[...], 0.01)
        pre_ref[...] = pre

        @pl.when(i == 0)
        def _():
            s1_ref[...] = jnp.zeros_like(s1_ref)
            s2_ref[...] = jnp.zeros_like(s2_ref)

        s1_ref[...] += jnp.sum(pre, axis=0, keepdims=True)
        s2_ref[...] += jnp.sum(pre * pre, axis=0, keepdims=True)

    return pl.pallas_call(
        body,
        grid=grid,
        in_specs=[
            pl.BlockSpec((p_parts, TN, CHUNK), lambda i: (0, i, 0)),
            pl.BlockSpec((TN, din), lambda i: (i, 0)),
            pl.BlockSpec((p_parts, CHUNK, HID), lambda i: (0, 0, 0)),
            pl.BlockSpec((din, HID), lambda i: (0, 0)),
            pl.BlockSpec((1, HID), lambda i: (0, 0)),
        ],
        out_specs=[
            pl.BlockSpec((TN, HID), lambda i: (i, 0)),
            pl.BlockSpec((1, HID), lambda i: (0, 0)),
            pl.BlockSpec((1, HID), lambda i: (0, 0)),
        ],
        out_shape=[
            jax.ShapeDtypeStruct((n, HID), jnp.float32),
            jax.ShapeDtypeStruct((1, HID), jnp.float32),
            jax.ShapeDtypeStruct((1, HID), jnp.float32),
        ],
    )(agg, h, wrt, wot, b)


def _norm_tc(pre, s1, s2, gamma, beta, n, emit_chunks):
    nc = HID // CHUNK

    def body(pre_ref, s1_ref, s2_ref, g_ref, b_ref, hn_ref, *maybe_hc):
        mu = s1_ref[...] * (1.0 / n)
        var = s2_ref[...] * (1.0 / n) - mu * mu
        a = g_ref[...] * lax.rsqrt(var + 1e-5)
        c = b_ref[...] - mu * a
        hn = pre_ref[...] * a + c
        hn_ref[...] = hn
        if emit_chunks:
            hc_ref = maybe_hc[0]
            for j in range(nc):
                hc_ref[j] = hn[:, j * CHUNK:(j + 1) * CHUNK]

    out_specs = [pl.BlockSpec((TN, HID), lambda i: (i, 0))]
    out_shape = [jax.ShapeDtypeStruct((n, HID), jnp.float32)]
    if emit_chunks:
        out_specs.append(pl.BlockSpec((nc, TN, CHUNK), lambda i: (0, i, 0)))
        out_shape.append(jax.ShapeDtypeStruct((nc, n, CHUNK), jnp.float32))

    return pl.pallas_call(
        body,
        grid=(n // TN,),
        in_specs=[
            pl.BlockSpec((TN, HID), lambda i: (i, 0)),
            pl.BlockSpec((1, HID), lambda i: (0, 0)),
            pl.BlockSpec((1, HID), lambda i: (0, 0)),
            pl.BlockSpec((1, HID), lambda i: (0, 0)),
            pl.BlockSpec((1, HID), lambda i: (0, 0)),
        ],
        out_specs=out_specs,
        out_shape=out_shape,
    )(pre, s1, s2, gamma, beta)


def _norm_mix_tc(pre2, s1, s2, gamma, beta, h0, h1, n):
    def body(pre_ref, s1_ref, s2_ref, g_ref, b_ref, h0_ref, h1_ref, out_ref):
        mu = s1_ref[...] * (1.0 / n)
        var = s2_ref[...] * (1.0 / n) - mu * mu
        a = g_ref[...] * lax.rsqrt(var + 1e-5)
        c = b_ref[...] - mu * a
        h2 = pre_ref[...] * a + c
        out_ref[...] = 0.7 * h0_ref[...] + 0.2 * h1_ref[...] + 0.1 * h2

    return pl.pallas_call(
        body,
        grid=(n // TN,),
        in_specs=[
            pl.BlockSpec((TN, HID), lambda i: (i, 0)),
            pl.BlockSpec((1, HID), lambda i: (0, 0)),
            pl.BlockSpec((1, HID), lambda i: (0, 0)),
            pl.BlockSpec((1, HID), lambda i: (0, 0)),
            pl.BlockSpec((1, HID), lambda i: (0, 0)),
            pl.BlockSpec((TN, HID), lambda i: (i, 0)),
            pl.BlockSpec((TN, HID), lambda i: (i, 0)),
        ],
        out_specs=pl.BlockSpec((TN, HID), lambda i: (i, 0)),
        out_shape=jax.ShapeDtypeStruct((n, HID), jnp.float32),
    )(pre2, s1, s2, gamma, beta, h0, h1)


def _mlp_tc(x0, x1, w1a, w1b, b1, w2, b2, w3, b3, w4, b4):
    b = x0.shape[0]

    def body(x0_ref, x1_ref, w1a_ref, w1b_ref, b1_ref, w2_ref, b2_ref,
             w3_ref, b3_ref, w4_ref, b4_ref, out_ref):
        bf = jnp.bfloat16
        h1 = jnp.dot(x0_ref[...].astype(bf), w1a_ref[...].astype(bf),
                     preferred_element_type=jnp.float32)
        h1 += jnp.dot(x1_ref[...].astype(bf), w1b_ref[...].astype(bf),
                      preferred_element_type=jnp.float32)
        h1 = jax.nn.leaky_relu(h1 + b1_ref[...], 0.01)
        h2 = jax.nn.leaky_relu(
            jnp.dot(h1.astype(bf), w2_ref[...].astype(bf),
                    preferred_element_type=jnp.float32)
            + b2_ref[...], 0.01)
        h3 = jax.nn.leaky_relu(
            jnp.dot(h2.astype(bf), w3_ref[...].astype(bf),
                    preferred_element_type=jnp.float32)
            + b3_ref[...], 0.01)
        out_ref[...] = (jnp.dot(h3, w4_ref[...],
                                preferred_element_type=jnp.float32)
                        + b4_ref[...])

    d1, d2, d3, d4 = 2048, 1024, 512, 2
    return pl.pallas_call(
        body,
        grid=(b // TB,),
        in_specs=[
            pl.BlockSpec((TB, HID), lambda i: (i, 0)),
            pl.BlockSpec((TB, HID), lambda i: (i, 0)),
            pl.BlockSpec((HID, d1), lambda i: (0, 0)),
            pl.BlockSpec((HID, d1), lambda i: (0, 0)),
            pl.BlockSpec((1, d1), lambda i: (0, 0)),
            pl.BlockSpec((d1, d2), lambda i: (0, 0)),
            pl.BlockSpec((1, d2), lambda i: (0, 0)),
            pl.BlockSpec((d2, d3), lambda i: (0, 0)),
            pl.BlockSpec((1, d3), lambda i: (0, 0)),
            pl.BlockSpec((d3, d4), lambda i: (0, 0)),
            pl.BlockSpec((1, d4), lambda i: (0, 0)),
        ],
        out_specs=pl.BlockSpec((TB, d4), lambda i: (i, 0)),
        out_shape=jax.ShapeDtypeStruct((b, d4), jnp.float32),
    )(x0, x1, w1a, w1b, b1, w2, b2, w3, b3, w4, b4)


def _pad_edges(ei, n):
    e = ei.shape[1]
    unit = 32 * EB * SB
    e_pad = -(-e // unit) * unit
    pad = e_pad - e
    r = jnp.arange(pad, dtype=jnp.int32)
    src = jnp.concatenate([ei[0], r % n]).reshape(e_pad // EB, EB)
    dst = jnp.concatenate([ei[1], n + (r % 16)]).reshape(e_pad // EB, EB)
    return src, dst, e_pad


def _graph_chain(x, src, dst, e_pad, params, prefix, zeros):
    p = params
    n = x.shape[0]
    hs = []
    c_chunks = 1
    h_flat = x
    h_mat = x
    for k in range(3):
        name = f"{prefix}{k}"
        w_rel = p["W_rel_" + name]
        w_root = p["W_root_" + name]
        bias = p["b_" + name].reshape(1, HID)
        gamma = p["gamma_" + name].reshape(1, HID)
        beta = p["beta_" + name].reshape(1, HID)

        agg = _segsum_sc(h_flat, src, dst, zeros, n, c_chunks, e_pad)
        p_parts = 2 if c_chunks == 1 else c_chunks
        if c_chunks == 1:
            wrt_s = jnp.stack([w_rel.T, w_rel.T])
        else:
            wrt_s = w_rel.T.reshape(p_parts, CHUNK, HID)
        agg_r = agg.reshape(p_parts, n, CHUNK)
        pre, s1, s2 = _layer_tc(agg_r, h_mat, wrt_s, w_root.T, bias)
        if k < 2:
            h_mat, hc = _norm_tc(pre, s1, s2, gamma, beta, n, True)
            c_chunks = HID // CHUNK
            h_flat = hc.reshape(c_chunks * n, CHUNK)
            hs.append(h_mat)
        else:
            out = _norm_mix_tc(pre, s1, s2, gamma, beta, hs[0], hs[1], n)
    return out


def kernel(gene_x, disease_x, params, gene_edge_index, disease_edge_index,
           batch_idx):
    p = params
    n = gene_x.shape[0]
    zeros = jnp.zeros((n + 16, CHUNK), jnp.float32)
    gsrc, gdst, e_g = _pad_edges(gene_edge_index, n)
    dsrc, ddst, e_d = _pad_edges(disease_edge_index, n)

    gene_out = _graph_chain(gene_x, gsrc, gdst, e_g, p, "g", zeros)
    dis_out = _graph_chain(disease_x, dsrc, ddst, e_d, p, "d", zeros)

    bt = batch_idx.T
    i0 = bt[0]
    i1 = bt[1]
    x0, x1 = _pair_gather_sc(gene_out, dis_out, i0, i1, batch_idx.shape[0])

    w1t = p["W_lin1"].T
    w1a = w1t[:HID]
    w1b = w1t[HID:]
    return _mlp_tc(
        x0, x1, w1a, w1b, p["b_lin1"].reshape(1, -1),
        p["W_lin2"].T, p["b_lin2"].reshape(1, -1),
        p["W_lin3"].T, p["b_lin3"].reshape(1, -1),
        p["W_lin4"].T, p["b_lin4"].reshape(1, -1),
    )
